# Initial kernel scaffold; baseline (speedup 1.0000x reference)
import jax
import jax.numpy as jnp
from jax.experimental import pallas as pl


def kernel(x, W0, W1, W2, W3):
    raise NotImplementedError("write your pallas kernel here")


# speedup vs baseline: 2.1851x; 2.1851x over previous
import jax
import jax.numpy as jnp
from jax import lax
from jax.experimental import pallas as pl
from jax.experimental.pallas import tpu as pltpu
from jax.experimental.pallas import tpu_sc as plsc

N = 100000
HIDDEN = 128
CHUNK = 128
NCHUNKS = (N + CHUNK - 1) // CHUNK
NPAD = NCHUNKS * CHUNK
NC = 2
NS = 16
NW = NC * NS
LANES = 16


def _sc_body(x0, x1, x2, x3, t0, t1, t2, t3, out,
             idx0, idx1, idx2, idx3, b0, b1, b2, b3, ob, sem):
    wid = lax.axis_index("s") * NC + lax.axis_index("c")
    extra = NCHUNKS % NW
    nmine = jnp.where(wid < extra, NCHUNKS // NW + 1, NCHUNKS // NW)

    def chunk_body(k, carry):
        base = pl.multiple_of((wid + NW * k) * CHUNK, CHUNK)
        pltpu.sync_copy(x0.at[pl.ds(base, CHUNK)], idx0)
        pltpu.sync_copy(x1.at[pl.ds(base, CHUNK)], idx1)
        pltpu.sync_copy(x2.at[pl.ds(base, CHUNK)], idx2)
        pltpu.sync_copy(x3.at[pl.ds(base, CHUNK)], idx3)
        cps = [pltpu.async_copy(t.at[i], b, sem)
               for t, i, b in ((t0, idx0, b0), (t1, idx1, b1),
                               (t2, idx2, b2), (t3, idx3, b3))]
        for cp in cps:
            cp.wait()

        def add_row(r, c2):
            for j in range(HIDDEN // LANES):
                s = pl.ds(j * LANES, LANES)
                ob[r, s] = b0[r, s] + b1[r, s] + b2[r, s] + b3[r, s]
            return c2

        lax.fori_loop(0, CHUNK, add_row, 0)
        pltpu.sync_copy(ob, out.at[pl.ds(base, CHUNK)])
        return carry

    lax.fori_loop(0, nmine, chunk_body, 0)


def kernel(x, W0, W1, W2, W3):
    xT = jnp.pad(x.astype(jnp.int32).T, ((0, 0), (0, NPAD - N)))
    x0, x1, x2, x3 = xT[0], xT[1], xT[2], xT[3]
    mesh = plsc.VectorSubcoreMesh(core_axis_name="c", subcore_axis_name="s")
    f = pl.kernel(
        _sc_body,
        mesh=mesh,
        out_type=jax.ShapeDtypeStruct((NPAD, HIDDEN), jnp.float32),
        scratch_types=[
            pltpu.VMEM((CHUNK,), jnp.int32),
            pltpu.VMEM((CHUNK,), jnp.int32),
            pltpu.VMEM((CHUNK,), jnp.int32),
            pltpu.VMEM((CHUNK,), jnp.int32),
            pltpu.VMEM((CHUNK, HIDDEN), jnp.float32),
            pltpu.VMEM((CHUNK, HIDDEN), jnp.float32),
            pltpu.VMEM((CHUNK, HIDDEN), jnp.float32),
            pltpu.VMEM((CHUNK, HIDDEN), jnp.float32),
            pltpu.VMEM((CHUNK, HIDDEN), jnp.float32),
            pltpu.SemaphoreType.DMA,
        ],
    )
    outp = f(x0, x1, x2, x3, W0, W1, W2, W3)
    return outp[:N]


# speedup vs baseline: 3.9304x; 1.7987x over previous
import jax
import jax.numpy as jnp
from jax import lax
from jax.experimental import pallas as pl
from jax.experimental.pallas import tpu as pltpu
from jax.experimental.pallas import tpu_sc as plsc

N = 100000
HIDDEN = 128
VOCAB = 64
CHUNK = 128
NCHUNKS = (N + CHUNK - 1) // CHUNK
NPAD = NCHUNKS * CHUNK
NC = 2
NS = 16
NW = NC * NS
LANES = 16


def _pair_body(w0, w1, w2, w3, t01, t23):
    t01[...] = w0[...][:, None, :] + w1[...][None, :, :]
    t23[...] = w2[...][:, None, :] + w3[...][None, :, :]


def _build_pair_tables(W0, W1, W2, W3):
    t01, t23 = pl.pallas_call(
        _pair_body,
        out_shape=[
            jax.ShapeDtypeStruct((VOCAB, VOCAB, HIDDEN), jnp.float32),
            jax.ShapeDtypeStruct((VOCAB, VOCAB, HIDDEN), jnp.float32),
        ],
    )(W0, W1, W2, W3)
    return (t01.reshape(VOCAB * VOCAB, HIDDEN),
            t23.reshape(VOCAB * VOCAB, HIDDEN))


def _sc_body(x0, x1, x2, x3, t01, t23, out,
             xa, xb, i01, i23, b01, b23, sem):
    wid = lax.axis_index("s") * NC + lax.axis_index("c")
    extra = NCHUNKS % NW
    nmine = jnp.where(wid < extra, NCHUNKS // NW + 1, NCHUNKS // NW)

    def chunk_body(k, carry):
        base = pl.multiple_of((wid + NW * k) * CHUNK, CHUNK)
        pltpu.sync_copy(x0.at[pl.ds(base, CHUNK)], xa)
        pltpu.sync_copy(x1.at[pl.ds(base, CHUNK)], xb)
        for j in range(CHUNK // LANES):
            s = pl.ds(j * LANES, LANES)
            i01[s] = xa[s] * VOCAB + xb[s]
        pltpu.sync_copy(x2.at[pl.ds(base, CHUNK)], xa)
        pltpu.sync_copy(x3.at[pl.ds(base, CHUNK)], xb)
        for j in range(CHUNK // LANES):
            s = pl.ds(j * LANES, LANES)
            i23[s] = xa[s] * VOCAB + xb[s]
        cp0 = pltpu.async_copy(t01.at[i01], b01, sem)
        cp1 = pltpu.async_copy(t23.at[i23], b23, sem)
        cp0.wait()
        cp1.wait()

        def add_row(r, c2):
            for j in range(HIDDEN // LANES):
                s = pl.ds(j * LANES, LANES)
                b01[r, s] = b01[r, s] + b23[r, s]
            return c2

        lax.fori_loop(0, CHUNK, add_row, 0)
        pltpu.sync_copy(b01, out.at[pl.ds(base, CHUNK)])
        return carry

    lax.fori_loop(0, nmine, chunk_body, 0)


def kernel(x, W0, W1, W2, W3):
    xT = jnp.pad(x.astype(jnp.int32).T, ((0, 0), (0, NPAD - N)))
    x0, x1, x2, x3 = xT[0], xT[1], xT[2], xT[3]
    t01, t23 = _build_pair_tables(W0, W1, W2, W3)
    mesh = plsc.VectorSubcoreMesh(core_axis_name="c", subcore_axis_name="s")
    f = pl.kernel(
        _sc_body,
        mesh=mesh,
        out_type=jax.ShapeDtypeStruct((NPAD, HIDDEN), jnp.float32),
        scratch_types=[
            pltpu.VMEM((CHUNK,), jnp.int32),
            pltpu.VMEM((CHUNK,), jnp.int32),
            pltpu.VMEM((CHUNK,), jnp.int32),
            pltpu.VMEM((CHUNK,), jnp.int32),
            pltpu.VMEM((CHUNK, HIDDEN), jnp.float32),
            pltpu.VMEM((CHUNK, HIDDEN), jnp.float32),
            pltpu.SemaphoreType.DMA,
        ],
    )
    outp = f(x0, x1, x2, x3, t01, t23)
    return outp[:N]
